# baseline (speedup 1.0000x reference)
import functools

import jax
import jax.numpy as jnp
from jax import lax
from jax.experimental import pallas as pl
from jax.experimental.pallas import tpu as pltpu
from jax.experimental.pallas import tpu_sc as plsc

B = 16384
N_DENSE = 13
N_SPARSE = 26
VOCAB = 100000
EMB_DIM = 16

NC = 2
NS = 16
NW = NC * NS

BPT = B // NW
RPT = BPT * N_SPARSE
CB = 4
CROWS = CB * N_SPARSE
NCHUNK = BPT // CB
NBUF = 4
OFF_PERIOD = 208
OFF_VECS = OFF_PERIOD // 16


def _sc_body(table_hbm, idx_hbm, offs_hbm, dv_hbm, dw_hbm, out_hbm,
             idx_v, offs_v, dv_v, dw_v, out_v, acc_v,
             buf0, buf1, buf2, buf3, sem0, sem1, sem2, sem3):
    bufs = (buf0, buf1, buf2, buf3)
    sems = (sem0, sem1, sem2, sem3)
    wid = lax.axis_index("s") * NC + lax.axis_index("c")
    ibase = wid * RPT
    bbase = wid * BPT

    pltpu.sync_copy(idx_hbm.at[pl.ds(ibase, RPT)], idx_v)
    pltpu.sync_copy(offs_hbm, offs_v)
    pltpu.sync_copy(dw_hbm, dw_v)
    pltpu.sync_copy(dv_hbm.at[pl.ds(bbase * EMB_DIM, BPT * EMB_DIM)], dv_v)

    def off_body(o, carry):
        for j in range(OFF_VECS):
            sl = pl.ds((o * OFF_VECS + j) * 16, 16)
            idx_v[sl] = idx_v[sl] + offs_v[pl.ds(j * 16, 16)]
        return carry

    lax.fori_loop(0, RPT // OFF_PERIOD, off_body, 0)

    def fire(chunk, slot):
        pltpu.make_async_copy(
            table_hbm.at[idx_v.at[pl.ds(chunk * CROWS, CROWS)]],
            bufs[slot], sems[slot]).start()

    def drain(chunk, slot):
        pltpu.make_async_copy(
            table_hbm.at[idx_v.at[pl.ds(chunk * CROWS, CROWS)]],
            bufs[slot], sems[slot]).wait()

    for s in range(NBUF):
        fire(s, s)

    dwv = dw_v[...]

    lanes16 = lax.iota(jnp.int32, 16) * 16

    def chunk_body(g, carry):
        for s in range(NBUF):
            c = g * NBUF + s
            drain(c, s)
            buf = bufs[s]
            for j in range(CB):
                bl = c * CB + j
                r0 = j * N_SPARSE
                acc0 = buf[r0 + 0, :]
                acc1 = buf[r0 + 1, :]
                acc2 = buf[r0 + 2, :]
                acc3 = dv_v[pl.ds(bl * EMB_DIM, EMB_DIM)] * dwv
                for r in range(3, N_SPARSE, 4):
                    acc0 = acc0 + buf[r0 + r, :]
                    acc1 = acc1 + buf[r0 + r + 1, :]
                    acc2 = acc2 + buf[r0 + r + 2, :]
                    if r + 3 < N_SPARSE:
                        acc3 = acc3 + buf[r0 + r + 3, :]
                acc_v[pl.ds((s * CB + j) * 16, 16)] = (
                    (acc0 + acc1) + (acc2 + acc3))

            @pl.when(g < NCHUNK // NBUF - 1)
            def _():
                fire(c + NBUF, s)

        r0 = plsc.load_gather(acc_v, [lanes16 + 0])
        r1 = plsc.load_gather(acc_v, [lanes16 + 1])
        r2 = plsc.load_gather(acc_v, [lanes16 + 2])
        r3 = plsc.load_gather(acc_v, [lanes16 + 3])
        for d in range(4, 16, 4):
            r0 = r0 + plsc.load_gather(acc_v, [lanes16 + d])
            r1 = r1 + plsc.load_gather(acc_v, [lanes16 + d + 1])
            r2 = r2 + plsc.load_gather(acc_v, [lanes16 + d + 2])
            r3 = r3 + plsc.load_gather(acc_v, [lanes16 + d + 3])
        out_v[pl.ds(g * 16, 16)] = (r0 + r1) + (r2 + r3)
        return carry

    lax.fori_loop(0, NCHUNK // NBUF, chunk_body, 0)

    pltpu.sync_copy(out_v, out_hbm.at[pl.ds(bbase, BPT)])


@functools.partial(
    pl.kernel,
    out_type=jax.ShapeDtypeStruct((B,), jnp.float32),
    mesh=plsc.VectorSubcoreMesh(core_axis_name="c", subcore_axis_name="s"),
    compiler_params=pltpu.CompilerParams(
        needs_layout_passes=False, use_tc_tiling_on_sc=False),
    scratch_types=[
        pltpu.VMEM((RPT,), jnp.int32),
        pltpu.VMEM((OFF_PERIOD,), jnp.int32),
        pltpu.VMEM((BPT * EMB_DIM,), jnp.float32),
        pltpu.VMEM((EMB_DIM,), jnp.float32),
        pltpu.VMEM((BPT,), jnp.float32),
        pltpu.VMEM((256,), jnp.float32),
        pltpu.VMEM((CROWS, EMB_DIM), jnp.float32),
        pltpu.VMEM((CROWS, EMB_DIM), jnp.float32),
        pltpu.VMEM((CROWS, EMB_DIM), jnp.float32),
        pltpu.VMEM((CROWS, EMB_DIM), jnp.float32),
        pltpu.SemaphoreType.DMA,
        pltpu.SemaphoreType.DMA,
        pltpu.SemaphoreType.DMA,
        pltpu.SemaphoreType.DMA,
    ],
)
def _sc_linear(table_hbm, idx_hbm, offs_hbm, dv_hbm, dw_hbm, out_hbm,
               idx_v, offs_v, dv_v, dw_v, out_v, acc_v,
               buf0, buf1, buf2, buf3, sem0, sem1, sem2, sem3):
    _sc_body(table_hbm, idx_hbm, offs_hbm, dv_hbm, dw_hbm, out_hbm,
             idx_v, offs_v, dv_v, dw_v, out_v, acc_v,
             buf0, buf1, buf2, buf3, sem0, sem1, sem2, sem3)


def kernel(inputs, emb_tables, dense_weight, bias):
    idx = inputs[:, N_DENSE:N_DENSE + N_SPARSE].astype(jnp.int32).reshape(-1)
    table = emb_tables.reshape(N_SPARSE * VOCAB, EMB_DIM)
    offs = ((jnp.arange(OFF_PERIOD, dtype=jnp.int32) % N_SPARSE)
            * jnp.int32(VOCAB))
    dv = jnp.concatenate(
        [inputs[:, :N_DENSE],
         jnp.ones((B, 1), jnp.float32),
         jnp.zeros((B, EMB_DIM - N_DENSE - 1), jnp.float32)], axis=1
    ).reshape(-1)
    dw = jnp.concatenate(
        [dense_weight[:, 0], bias,
         jnp.zeros((EMB_DIM - N_DENSE - 1,), jnp.float32)])
    out = _sc_linear(table, idx, offs, dv, dw)
    return out.reshape(B, 1)
